# Initial kernel scaffold; baseline (speedup 1.0000x reference)
import jax
import jax.numpy as jnp
from jax.experimental import pallas as pl


def kernel(atom_num, dis1, dis2, id1u, id1v, id2u, id2v, params):
    raise NotImplementedError("write your pallas kernel here")


# speedup vs baseline: 3.4703x; 3.4703x over previous
import functools

import jax
import jax.numpy as jnp
from jax import lax
from jax.experimental import pallas as pl
from jax.experimental.pallas import tpu as pltpu
from jax.experimental.pallas import tpu_sc as plsc

HD = 128
N = 10000
E = 320000
LANES = 16

NB = 1000
BE = 2000
KMIC = 80
NMICRO = 5
KMAC = KMIC * NMICRO
KMICS = 40
KMACS = KMICS * NMICRO
NC = 2
NS = 16
NW = NC * NS
EPW = E // NW
NMAC = EPW // KMAC
NMACS = EPW // KMACS
ROWS_PT = 624
ROWS_LAST = N - (NS - 1) * ROWS_PT


def _silu(x):
    return x * lax.logistic(x)


def _prep_body(an, emb, w1u, w1v, b1, w2u, w2v, b2,
               h_o, pu1_o, pv1_o, pu2_o, pv2_o):
    iota = lax.broadcasted_iota(jnp.int32, (1, HD), 1)
    oh = (an[...] == iota).astype(jnp.float32)
    h = jnp.dot(oh, emb[...], preferred_element_type=jnp.float32)
    h_o[...] = h
    pu1_o[...] = jnp.dot(h, w1u[...], preferred_element_type=jnp.float32) + b1[...]
    pv1_o[...] = jnp.dot(h, w1v[...], preferred_element_type=jnp.float32)
    pu2_o[...] = jnp.dot(h, w2u[...], preferred_element_type=jnp.float32) + b2[...]
    pv2_o[...] = jnp.dot(h, w2v[...], preferred_element_type=jnp.float32)


def _tc_prep(atom2d, emb_pad, w1u, w1v, b1, w2u, w2v, b2):
    row = pl.BlockSpec((NB, HD), lambda i: (i, 0))
    wsp = pl.BlockSpec((HD, HD), lambda i: (0, 0))
    bsp = pl.BlockSpec((1, HD), lambda i: (0, 0))
    return pl.pallas_call(
        _prep_body,
        grid=(N // NB,),
        in_specs=[pl.BlockSpec((NB, 1), lambda i: (i, 0)),
                  wsp, wsp, wsp, bsp, wsp, wsp, bsp],
        out_specs=[row] * 5,
        out_shape=[jax.ShapeDtypeStruct((N, HD), jnp.float32)] * 5,
    )(atom2d, emb_pad, w1u, w1v, b1, w2u, w2v, b2)


def _emlp_body(g, dis, w1d, w2, b2, w3, b3, m_o):
    x = _silu(g[...] + dis[...] * w1d[...])
    x = _silu(jnp.dot(x, w2[...], preferred_element_type=jnp.float32) + b2[...])
    m_o[...] = jnp.dot(x, w3[...], preferred_element_type=jnp.float32) + b3[...]


def _tc_emlp(g, dis2d, w1d, w2, b2, w3, b3):
    row = pl.BlockSpec((BE, HD), lambda i: (i, 0))
    wsp = pl.BlockSpec((HD, HD), lambda i: (0, 0))
    bsp = pl.BlockSpec((1, HD), lambda i: (0, 0))
    return pl.pallas_call(
        _emlp_body,
        grid=(E // BE,),
        in_specs=[row, pl.BlockSpec((BE, 1), lambda i: (i, 0)),
                  bsp, wsp, bsp, wsp, bsp],
        out_specs=row,
        out_shape=jax.ShapeDtypeStruct((E, HD), jnp.float32),
    )(g, dis2d, w1d, w2, b2, w3, b3)


def _upd_common(h, s1, s2, uh, ua, ub, b1, w2, b2):
    a1 = s1[0] + s1[1]
    a2 = s2[0] + s2[1]
    x = _silu(jnp.dot(h[...], uh[...], preferred_element_type=jnp.float32)
              + jnp.dot(a1, ua[...], preferred_element_type=jnp.float32)
              + jnp.dot(a2, ub[...], preferred_element_type=jnp.float32)
              + b1[...])
    x = jnp.dot(x, w2[...], preferred_element_type=jnp.float32) + b2[...]
    return h[...] + x


def _upd1_body(h, s1, s2, uh, ua, ub, b1, w2, b2,
               qw1u, qw1v, qb1, qw2u, qw2v, qb2,
               h_o, pu1_o, pv1_o, pu2_o, pv2_o):
    hn = _upd_common(h, s1, s2, uh, ua, ub, b1, w2, b2)
    h_o[...] = hn
    pu1_o[...] = jnp.dot(hn, qw1u[...], preferred_element_type=jnp.float32) + qb1[...]
    pv1_o[...] = jnp.dot(hn, qw1v[...], preferred_element_type=jnp.float32)
    pu2_o[...] = jnp.dot(hn, qw2u[...], preferred_element_type=jnp.float32) + qb2[...]
    pv2_o[...] = jnp.dot(hn, qw2v[...], preferred_element_type=jnp.float32)


def _tc_upd1(h, s1, s2, uh, ua, ub, b1, w2, b2,
             qw1u, qw1v, qb1, qw2u, qw2v, qb2):
    row = pl.BlockSpec((NB, HD), lambda i: (i, 0))
    ssp = pl.BlockSpec((NC, NB, HD), lambda i: (0, i, 0))
    wsp = pl.BlockSpec((HD, HD), lambda i: (0, 0))
    bsp = pl.BlockSpec((1, HD), lambda i: (0, 0))
    return pl.pallas_call(
        _upd1_body,
        grid=(N // NB,),
        in_specs=[row, ssp, ssp, wsp, wsp, wsp, bsp, wsp, bsp,
                  wsp, wsp, bsp, wsp, wsp, bsp],
        out_specs=[row] * 5,
        out_shape=[jax.ShapeDtypeStruct((N, HD), jnp.float32)] * 5,
    )(h, s1, s2, uh, ua, ub, b1, w2, b2, qw1u, qw1v, qb1, qw2u, qw2v, qb2)


def _upd2_body(h, s1, s2, uh, ua, ub, b1, w2, b2, wout, bout, d_o):
    hn = _upd_common(h, s1, s2, uh, ua, ub, b1, w2, b2)
    d_o[...] = jnp.dot(hn, wout[...], preferred_element_type=jnp.float32) + bout[...]


def _tc_upd2(h, s1, s2, uh, ua, ub, b1, w2, b2, wout_pad, bout_pad):
    row = pl.BlockSpec((NB, HD), lambda i: (i, 0))
    ssp = pl.BlockSpec((NC, NB, HD), lambda i: (0, i, 0))
    wsp = pl.BlockSpec((HD, HD), lambda i: (0, 0))
    bsp = pl.BlockSpec((1, HD), lambda i: (0, 0))
    return pl.pallas_call(
        _upd2_body,
        grid=(N // NB,),
        in_specs=[row, ssp, ssp, wsp, wsp, wsp, bsp, wsp, bsp,
                  pl.BlockSpec((HD, 8), lambda i: (0, 0)),
                  pl.BlockSpec((1, 8), lambda i: (0, 0))],
        out_specs=pl.BlockSpec((NB, 8), lambda i: (i, 0)),
        out_shape=jax.ShapeDtypeStruct((N, 8), jnp.float32),
    )(h, s1, s2, uh, ua, ub, b1, w2, b2, wout_pad, bout_pad)


@functools.lru_cache(maxsize=None)
def _sc_gather_fn():
    mesh = plsc.VectorSubcoreMesh(core_axis_name="c", subcore_axis_name="s")

    @functools.partial(
        pl.kernel, mesh=mesh,
        out_type=jax.ShapeDtypeStruct((E, HD), jnp.float32),
        scratch_types=[
            pltpu.VMEM((KMAC,), jnp.int32),
            pltpu.VMEM((KMAC,), jnp.int32),
            pltpu.VMEM((KMAC, HD), jnp.float32),
            pltpu.VMEM((KMAC, HD), jnp.float32),
            pltpu.SemaphoreType.DMA,
            pltpu.SemaphoreType.DMA,
        ])
    def _gather(pu_hbm, pv_hbm, iu_hbm, iv_hbm, out_hbm,
                iu_v, iv_v, a_v, b_v, sem_a, sem_b):
        wid = lax.axis_index("s") * NC + lax.axis_index("c")
        ebase = wid * EPW

        def mac(ci, carry):
            e0 = ebase + ci * KMAC
            pltpu.sync_copy(iu_hbm.at[pl.ds(e0, KMAC)], iu_v)
            pltpu.sync_copy(iv_hbm.at[pl.ds(e0, KMAC)], iv_v)
            cps = []
            for j in range(NMICRO):
                sl = pl.ds(j * KMIC, KMIC)
                cps.append(pltpu.async_copy(pu_hbm.at[iu_v.at[sl]], a_v.at[sl], sem_a))
                cps.append(pltpu.async_copy(pv_hbm.at[iv_v.at[sl]], b_v.at[sl], sem_b))
            for cp in cps:
                cp.wait()

            def add_row(e, c2):
                for j in range(HD // LANES):
                    sl = pl.ds(j * LANES, LANES)
                    a_v[e, sl] = a_v[e, sl] + b_v[e, sl]
                return c2

            lax.fori_loop(0, KMAC, add_row, 0)
            pltpu.sync_copy(a_v, out_hbm.at[pl.ds(ebase + ci * KMAC, KMAC)])
            return carry

        lax.fori_loop(0, NMAC, mac, 0)

    return _gather


def _sc_gather(pu, pv, iu, iv):
    return _sc_gather_fn()(pu, pv, iu, iv)


@functools.lru_cache(maxsize=None)
def _sc_scatter_fn():
    mesh = plsc.VectorSubcoreMesh(core_axis_name="c", subcore_axis_name="s")

    @functools.partial(
        pl.kernel, mesh=mesh,
        out_type=jax.ShapeDtypeStruct((NC, N, HD), jnp.float32),
        scratch_types=[
            pltpu.VMEM((NMICRO, KMICS), jnp.int32),
            pltpu.VMEM((KMACS, HD), jnp.float32),
            pltpu.VMEM_SHARED((N, HD), jnp.float32),
        ])
    def _scatter(m_hbm, iv_hbm, z_hbm, out_hbm, iv_v, m_v, s_sh):
        c = lax.axis_index("c")
        s = lax.axis_index("s")
        wid = s * NC + c
        r0 = s * ROWS_PT

        @pl.when(s < NS - 1)
        def _():
            pltpu.sync_copy(z_hbm.at[pl.ds(r0, ROWS_PT)], s_sh.at[pl.ds(r0, ROWS_PT)])

        @pl.when(s == NS - 1)
        def _():
            pltpu.sync_copy(z_hbm.at[pl.ds(r0, ROWS_LAST)], s_sh.at[pl.ds(r0, ROWS_LAST)])

        plsc.subcore_barrier()
        ebase = wid * EPW

        def mac(ci, carry):
            mid = wid * NMACS + ci
            pltpu.sync_copy(iv_hbm.at[mid], iv_v)
            pltpu.sync_copy(m_hbm.at[pl.ds(ebase + ci * KMACS, KMACS)], m_v)
            for j in range(NMICRO):
                pltpu.sync_copy(m_v.at[pl.ds(j * KMICS, KMICS)],
                                s_sh.at[iv_v.at[j]], add=True)
            return carry

        lax.fori_loop(0, NMACS, mac, 0)
        plsc.subcore_barrier()

        @pl.when(s < NS - 1)
        def _():
            pltpu.sync_copy(s_sh.at[pl.ds(r0, ROWS_PT)],
                            out_hbm.at[c].at[pl.ds(r0, ROWS_PT)])

        @pl.when(s == NS - 1)
        def _():
            pltpu.sync_copy(s_sh.at[pl.ds(r0, ROWS_LAST)],
                            out_hbm.at[c].at[pl.ds(r0, ROWS_LAST)])

    return _scatter


def _sc_scatter(m, iv, z):
    return _sc_scatter_fn()(m, iv, z)


def _split_edge_w(ep):
    w1 = ep['W1']
    return (w1[:HD], w1[HD:2 * HD], w1[2 * HD:2 * HD + 1],
            ep['b1'].reshape(1, HD), ep['W2'], ep['b2'].reshape(1, HD),
            ep['W3'], ep['b3'].reshape(1, HD))


def _split_upd_w(up):
    w1 = up['W1']
    return (w1[:HD], w1[HD:2 * HD], w1[2 * HD:], up['b1'].reshape(1, HD),
            up['W2'], up['b2'].reshape(1, HD))


def kernel(atom_num, dis1, dis2, id1u, id1v, id2u, id2v, params):
    p = params
    atom2d = atom_num.astype(jnp.int32).reshape(N, 1)
    dis1_2d = dis1.reshape(E, 1)
    dis2_2d = dis2.reshape(E, 1)
    i1u = id1u.astype(jnp.int32)
    i1v = id1v.astype(jnp.int32)
    i2u = id2u.astype(jnp.int32)
    i2v = id2v.astype(jnp.int32)
    i1v3 = i1v.reshape(E // KMACS, NMICRO, KMICS)
    i2v3 = i2v.reshape(E // KMACS, NMICRO, KMICS)
    z = jnp.zeros((N, HD), jnp.float32)

    emb = p['atom_emb']
    emb_pad = jnp.pad(emb, ((0, HD - emb.shape[0]), (0, 0)))

    e1u, e1v, e1d, e1b1, e1w2, e1b2, e1w3, e1b3 = _split_edge_w(p['edge1'])
    e2u, e2v, e2d, e2b1, e2w2, e2b2, e2w3, e2b3 = _split_edge_w(p['edge2'])
    f1u, f1v, f1d, f1b1, f1w2, f1b2, f1w3, f1b3 = _split_edge_w(p['uedge1'])
    f2u, f2v, f2d, f2b1, f2w2, f2b2, f2w3, f2b3 = _split_edge_w(p['uedge2'])
    u1h, u1a, u1b, u1b1, u1w2, u1b2 = _split_upd_w(p['upd1'])
    u2h, u2a, u2b, u2b1, u2w2, u2b2 = _split_upd_w(p['upd2'])

    wout_pad = jnp.pad(p['Wout'], ((0, 0), (0, 8 - p['Wout'].shape[1])))
    bout_pad = jnp.pad(p['bout'], (0, 8 - p['bout'].shape[0])).reshape(1, 8)

    h, pu1, pv1, pu2, pv2 = _tc_prep(atom2d, emb_pad, e1u, e1v, e1b1,
                                     e2u, e2v, e2b1)
    g1 = _sc_gather(pu1, pv1, i1u, i1v)
    g2 = _sc_gather(pu2, pv2, i2u, i2v)
    m1 = _tc_emlp(g1, dis1_2d, e1d, e1w2, e1b2, e1w3, e1b3)
    m2 = _tc_emlp(g2, dis2_2d, e2d, e2w2, e2b2, e2w3, e2b3)
    s1 = _sc_scatter(m1, i1v3, z)
    s2 = _sc_scatter(m2, i2v3, z)
    h1, qu1, qv1, qu2, qv2 = _tc_upd1(h, s1, s2, u1h, u1a, u1b, u1b1, u1w2,
                                      u1b2, f1u, f1v, f1b1, f2u, f2v, f2b1)

    g1 = _sc_gather(qu1, qv1, i1u, i1v)
    g2 = _sc_gather(qu2, qv2, i2u, i2v)
    m1 = _tc_emlp(g1, dis1_2d, f1d, f1w2, f1b2, f1w3, f1b3)
    m2 = _tc_emlp(g2, dis2_2d, f2d, f2w2, f2b2, f2w3, f2b3)
    s1 = _sc_scatter(m1, i1v3, z)
    s2 = _sc_scatter(m2, i2v3, z)
    delta8 = _tc_upd2(h1, s1, s2, u2h, u2a, u2b, u2b1, u2w2, u2b2,
                      wout_pad, bout_pad)
    return delta8[:, :3]
